# baseline (speedup 1.0000x reference)
import functools

import jax
import jax.numpy as jnp
from jax import lax
from jax.experimental import pallas as pl
from jax.experimental.pallas import tpu as pltpu
from jax.experimental.pallas import tpu_sc as plsc

_B = 4096
_L = 200
_D = 64
_NC = 2
_NS = 16
_NW = _NC * _NS
_RPW = _B // _NW
_ZROW = _RPW


def _emb_body(idx_hbm, seq_hbm, table_hbm,
              out_hbm,
              idxl_v, idxt_v, seq_v, rows_a, rows_b, out_a, out_b,
              sem_g, sem_w0, sem_w1):
    wid = lax.axis_index("s") * _NC + lax.axis_index("c")
    b0 = wid * _RPW
    pltpu.sync_copy(seq_hbm.at[pl.ds(b0, _RPW)], seq_v.at[pl.ds(0, _RPW)])
    pltpu.sync_copy(idx_hbm.at[pl.ds(b0 * _L, _RPW * _L)], idxl_v)

    zero16 = jnp.zeros((16,), jnp.float32)
    iota16 = lax.iota(jnp.int32, 16)
    for c4 in range(4):
        rows_a[_ZROW, pl.ds(c4 * 16, 16)] = zero16
        rows_b[_ZROW, pl.ds(c4 * 16, 16)] = zero16

    def tbody(l, carry):
        lsplat = jnp.full((16,), 0, jnp.int32) + l
        for jg in range(8):
            flat = (iota16 + (jg * 16)) * _L + lsplat
            vals = plsc.load_gather(idxl_v, [flat])
            idxt_v[l, pl.ds(jg * 16, 16)] = vals
        return carry
    lax.fori_loop(0, _L, tbody, 0)

    pltpu.async_copy(table_hbm.at[idxt_v.at[0]],
                     rows_a.at[pl.ds(0, _RPW), :], sem_g)

    def do_l(l, rows_v, out_v, sem_w, nxt_rows, j):
        @pl.when(j >= 1)
        def _():
            pltpu.make_async_copy(out_hbm.at[0, :, pl.ds(0, _RPW)],
                                  out_v, sem_w).wait()
        pltpu.make_async_copy(table_hbm.at[pl.ds(0, 128), :],
                              rows_v.at[pl.ds(0, _RPW), :], sem_g).wait()

        @pl.when(l + 1 < _L)
        def _():
            pltpu.async_copy(table_hbm.at[idxt_v.at[l + 1]],
                             nxt_rows.at[pl.ds(0, _RPW), :], sem_g)

        lsplat = jnp.full((16,), 0, jnp.int32) + l
        rowidx = []
        for jg in range(8):
            lens = seq_v[pl.ds(jg * 16, 16)]
            m = lsplat < lens
            rowidx.append(jnp.where(m, iota16 + (jg * 16),
                                    jnp.full((16,), _ZROW, jnp.int32)))
        for d in range(_D):
            dsplat = jnp.full((16,), d, jnp.int32)
            for jg in range(8):
                vals = plsc.load_gather(rows_v, [rowidx[jg], dsplat])
                out_v[d, pl.ds(jg * 16, 16)] = vals
        pltpu.async_copy(out_v, out_hbm.at[l, :, pl.ds(b0, _RPW)], sem_w)

    def body(j, carry):
        do_l(2 * j, rows_a, out_a, sem_w0, rows_b, j)
        do_l(2 * j + 1, rows_b, out_b, sem_w1, rows_a, j)
        return carry

    lax.fori_loop(0, _L // 2, body, 0)

    for sem_w, out_v in ((sem_w0, out_a), (sem_w1, out_b)):
        pltpu.make_async_copy(out_hbm.at[0, :, pl.ds(0, _RPW)],
                              out_v, sem_w).wait()


@jax.jit
def _emb_call(idx_flat, seq, table):
    mesh = plsc.VectorSubcoreMesh(core_axis_name="c", subcore_axis_name="s",
                                  num_cores=_NC, num_subcores=_NS)
    fn = pl.kernel(
        _emb_body,
        out_type=jax.ShapeDtypeStruct((_L, _D, _B), jnp.float32),
        mesh=mesh,
        scratch_types=[
            pltpu.VMEM((_RPW * _L,), jnp.int32),
            pltpu.VMEM((_L, _RPW), jnp.int32),
            pltpu.VMEM((_RPW + 16,), jnp.int32),
            pltpu.VMEM((_RPW + 1, _D), jnp.float32),
            pltpu.VMEM((_RPW + 1, _D), jnp.float32),
            pltpu.VMEM((_D, _RPW), jnp.float32),
            pltpu.VMEM((_D, _RPW), jnp.float32),
            pltpu.SemaphoreType.DMA,
            pltpu.SemaphoreType.DMA,
            pltpu.SemaphoreType.DMA,
        ],
        compiler_params=pltpu.CompilerParams(use_tc_tiling_on_sc=False,
                                             needs_layout_passes=False),
    )
    return fn(idx_flat, seq, table)


def kernel(indices, seq_lens, table):
    idx_flat = indices.reshape(_B * _L).astype(jnp.int32)
    seq = seq_lens.astype(jnp.int32)
    out_t = _emb_call(idx_flat, seq, table)
    out = jnp.transpose(out_t, (2, 0, 1))
    mask = (jnp.arange(_L, dtype=jnp.int32)[None, :]
            < seq_lens.astype(jnp.int32)[:, None]).astype(table.dtype)
    lengths = jnp.broadcast_to(mask[:, :, None], (_B, _L, _D))
    return out, lengths


# speedup vs baseline: 1.4868x; 1.4868x over previous
import functools

import jax
import jax.numpy as jnp
from jax import lax
from jax.experimental import pallas as pl
from jax.experimental.pallas import tpu as pltpu
from jax.experimental.pallas import tpu_sc as plsc

_B = 4096
_L = 200
_D = 64
_NC = 2
_NS = 16
_NW = _NC * _NS
_RPW = _B // _NW
_V = 1000000
_SIZES = (128, 64, 32, 16, 8, 4, 2, 1)


def _emb_body(idx_hbm, seq_hbm, table_hbm, zeros_hbm,
              out_hbm,
              idx_v, seq_v, rows_v, zeros_v,
              sem_g, sem_w0, sem_w1):
    wid = lax.axis_index("s") * _NC + lax.axis_index("c")
    base = wid * _RPW
    pltpu.sync_copy(seq_hbm.at[pl.ds(base, _RPW)], seq_v.at[pl.ds(0, _RPW)])
    pltpu.sync_copy(zeros_hbm, zeros_v)

    def do_row(i, slot, sem_w, j):
        b = base + i
        row0 = b * _L

        @pl.when(j >= 1)
        def _():
            pltpu.make_async_copy(table_hbm.at[pl.ds(0, _L), :],
                                  rows_v.at[slot], sem_w).wait()

        pltpu.sync_copy(idx_hbm.at[pl.ds(row0, _L)], idx_v)
        g1 = pltpu.async_copy(table_hbm.at[idx_v.at[pl.ds(0, 128)]],
                              rows_v.at[slot, pl.ds(0, 128), :], sem_g)
        g2 = pltpu.async_copy(table_hbm.at[idx_v.at[pl.ds(128, _L - 128)]],
                              rows_v.at[slot, pl.ds(128, _L - 128), :], sem_g)

        ln = seq_v[pl.ds(i, 16)][0]
        rem = _L - ln

        soff = ln
        for s in _SIZES:
            sbit = (rem & s) != 0

            @pl.when(sbit)
            def _(soff=soff, s=s):
                pltpu.async_copy(zeros_v.at[pl.ds(0, s), :],
                                 out_hbm.at[b, pl.ds(soff, s), :], sem_w)

            soff = soff + s * sbit.astype(jnp.int32)

        g1.wait()
        g2.wait()

        off = 0
        for s in _SIZES:
            pbit = (ln & s) != 0

            @pl.when(pbit)
            def _(off=off, s=s):
                pltpu.async_copy(rows_v.at[slot, pl.ds(off, s), :],
                                 out_hbm.at[b, pl.ds(off, s), :], sem_w)

            off = off + s * pbit.astype(jnp.int32)

    def body(j, carry):
        do_row(2 * j, 0, sem_w0, j)
        do_row(2 * j + 1, 1, sem_w1, j)
        return carry

    lax.fori_loop(0, _RPW // 2, body, 0)

    for sem_w, slot in ((sem_w0, 0), (sem_w1, 1)):
        pltpu.make_async_copy(table_hbm.at[pl.ds(0, _L), :],
                              rows_v.at[slot], sem_w).wait()


@jax.jit
def _emb_call(idx_flat, seq, table, zeros):
    mesh = plsc.VectorSubcoreMesh(core_axis_name="c", subcore_axis_name="s",
                                  num_cores=_NC, num_subcores=_NS)
    fn = pl.kernel(
        _emb_body,
        out_type=jax.ShapeDtypeStruct((_B, _L, _D), jnp.float32),
        mesh=mesh,
        scratch_types=[
            pltpu.VMEM((_L,), jnp.int32),
            pltpu.VMEM((_RPW + 16,), jnp.int32),
            pltpu.VMEM((2, _L, _D), jnp.float32),
            pltpu.VMEM((128, _D), jnp.float32),
            pltpu.SemaphoreType.DMA,
            pltpu.SemaphoreType.DMA,
            pltpu.SemaphoreType.DMA,
        ],
        compiler_params=pltpu.CompilerParams(use_tc_tiling_on_sc=False),
    )
    return fn(idx_flat, seq, table, zeros)


def kernel(indices, seq_lens, table):
    idx_flat = indices.reshape(_B * _L).astype(jnp.int32)
    seq = seq_lens.astype(jnp.int32)
    zeros = jnp.zeros((128, _D), jnp.float32)
    out = _emb_call(idx_flat, seq, table, zeros)
    mask = (jnp.arange(_L, dtype=jnp.int32)[None, :]
            < seq_lens.astype(jnp.int32)[:, None]).astype(table.dtype)
    lengths = jnp.broadcast_to(mask[:, :, None], (_B, _L, _D))
    return out, lengths
